# baseline (speedup 1.0000x reference)
import functools

import jax
import jax.numpy as jnp
from jax import lax
from jax.experimental import pallas as pl
from jax.experimental.pallas import tpu as pltpu
from jax.experimental.pallas import tpu_sc as plsc

NC = 2
NS = 16
NW = NC * NS
CHUNK = 128


def _sc_mesh():
    return plsc.VectorSubcoreMesh(
        core_axis_name="c", subcore_axis_name="s", num_cores=NC, num_subcores=NS
    )


def _make_gather(n, d, c_lo, c_hi):
    dp = d // 2
    ncr = c_hi - c_lo

    @functools.partial(
        pl.kernel,
        mesh=_sc_mesh(),
        out_type=jax.ShapeDtypeStruct((ncr * CHUNK, d), jnp.int32),
        scratch_types=[
            pltpu.VMEM((CHUNK,), jnp.int32),
            pltpu.VMEM((CHUNK,), jnp.int32),
            pltpu.VMEM((CHUNK, dp), jnp.int32),
            pltpu.VMEM((CHUNK, dp), jnp.int32),
            pltpu.SemaphoreType.DMA,
            pltpu.SemaphoreType.DMA,
        ],
        compiler_params=pltpu.CompilerParams(use_tc_tiling_on_sc=False),
    )
    def gather_k(x_hbm, src_hbm, dst_hbm, xg_out, sidx, didx, srows, drows, s1, s2):
        cid = lax.axis_index("c")
        sid = lax.axis_index("s")
        wid = sid * NC + cid
        w0 = (ncr * wid) // NW
        w1 = (ncr * (wid + 1)) // NW

        def body(c, _):
            base = (c_lo + c) * CHUNK
            pltpu.sync_copy(src_hbm.at[pl.ds(base, CHUNK)], sidx)
            pltpu.sync_copy(dst_hbm.at[pl.ds(base, CHUNK)], didx)
            cp1 = pltpu.async_copy(x_hbm.at[sidx], srows, s1)
            cp2 = pltpu.async_copy(x_hbm.at[didx], drows, s2)
            cp1.wait()
            cp2.wait()
            pltpu.sync_copy(srows, xg_out.at[pl.ds(c * CHUNK, CHUNK), pl.ds(0, dp)])
            pltpu.sync_copy(drows, xg_out.at[pl.ds(c * CHUNK, CHUNK), pl.ds(dp, dp)])
            return 0

        lax.fori_loop(w0, w1, body, 0)

    return gather_k


def _make_scatter(n_pad, d, c_lo, c_hi):
    rows_per_tile = n_pad // NS
    ncr = c_hi - c_lo

    @functools.partial(
        pl.kernel,
        mesh=_sc_mesh(),
        out_type=jax.ShapeDtypeStruct((NC, n_pad, d), jnp.float32),
        scratch_types=[
            pltpu.VMEM_SHARED((n_pad, d), jnp.float32),
            pltpu.VMEM((CHUNK,), jnp.int32),
            pltpu.VMEM((CHUNK, d), jnp.float32),
        ],
    )
    def scatter_k(m_hbm, dst_hbm, zero_hbm, acc_out, acc, didx, mrows):
        cid = lax.axis_index("c")
        sid = lax.axis_index("s")
        wid = sid * NC + cid
        w0 = (ncr * wid) // NW
        w1 = (ncr * (wid + 1)) // NW

        r0 = sid * rows_per_tile
        pltpu.sync_copy(zero_hbm.at[pl.ds(r0, rows_per_tile)], acc.at[pl.ds(r0, rows_per_tile)])
        plsc.subcore_barrier()

        def body(c, _):
            pltpu.sync_copy(dst_hbm.at[pl.ds((c_lo + c) * CHUNK, CHUNK)], didx)
            pltpu.sync_copy(m_hbm.at[pl.ds(c * CHUNK, CHUNK)], mrows)
            pltpu.sync_copy(mrows, acc.at[didx], add=True)
            return 0

        lax.fori_loop(w0, w1, body, 0)
        plsc.subcore_barrier()
        pltpu.sync_copy(acc.at[pl.ds(r0, rows_per_tile)], acc_out.at[cid, pl.ds(r0, rows_per_tile)])

    return scatter_k


def _mlp_body(nrows, be, mask_from, xg_ref, e_ref,
              awlo_ref, ab1_ref, aw2_ref, ab2_ref, aw3_ref, ab3_ref,
              mwlo_ref, mb1_ref, mw2_ref, mb2_ref, out_ref):
    xg = xg_ref[...]
    lo = jax.lax.bitcast_convert_type(xg << 16, jnp.float32).astype(jnp.bfloat16)
    hi = jax.lax.bitcast_convert_type(xg & jnp.int32(-65536), jnp.float32).astype(jnp.bfloat16)
    e = e_ref[...].astype(jnp.bfloat16)
    hcat = jnp.concatenate([lo, hi, e], axis=-1)
    dot = functools.partial(jnp.dot, preferred_element_type=jnp.float32)
    t = dot(hcat, awlo_ref[...]) + ab1_ref[...]
    t = jnp.maximum(t, 0.0).astype(jnp.bfloat16)
    t = jnp.maximum(dot(t, aw2_ref[...]) + ab2_ref[...], 0.0)
    a = dot(t.astype(jnp.bfloat16), aw3_ref[...]) + ab3_ref[...]
    h = dot(hcat, mwlo_ref[...]) + mb1_ref[...]
    h = jnp.maximum(h, 0.0).astype(jnp.bfloat16)
    m = dot(h, mw2_ref[...]) + mb2_ref[...]
    msg = jax.nn.sigmoid(a) * m
    if mask_from is not None:
        row = pl.program_id(0) * be + lax.broadcasted_iota(jnp.int32, msg.shape, 0)
        msg = jnp.where(row < mask_from, msg, 0.0)
    out_ref[...] = msg


def _combine_body(x_ref, *refs):
    arefs = refs[:-1]
    out_ref = refs[-1]
    acc = x_ref[...]
    for a_ref in arefs:
        acc = acc + a_ref[0] + a_ref[1]
    out_ref[...] = acc


def kernel(X, E, emb_nodes, emb_edges, edge_index,
           att_W1, att_b1, att_W2, att_b2, att_W3, att_b3,
           msg_W1, msg_b1, msg_W2, msg_b2):
    n, d = X.shape
    ne = E.shape[0]
    tch = -(-ne // CHUNK)
    ne_c = tch * CHUNK
    pad = ne_c - ne

    if pad:
        src = jnp.concatenate([edge_index[0], jnp.zeros((pad,), jnp.int32)])
        dst = jnp.concatenate([edge_index[1], jnp.zeros((pad,), jnp.int32)])
        e_in = jnp.concatenate([E, jnp.zeros((pad, d), jnp.float32)], axis=0)
    else:
        src = edge_index[0]
        dst = edge_index[1]
        e_in = E

    xb = X.astype(jnp.bfloat16)
    lo16 = jax.lax.bitcast_convert_type(xb[:, : d // 2], jnp.uint16).astype(jnp.uint32)
    hi16 = jax.lax.bitcast_convert_type(xb[:, d // 2:], jnp.uint16).astype(jnp.uint32)
    xpack = jax.lax.bitcast_convert_type((hi16 << 16) | lo16, jnp.int32)

    half = d // 2

    def wlo(w):
        return jnp.concatenate([w[:half], w[d:d + half]], axis=0).astype(jnp.bfloat16)

    def whi(w):
        return jnp.concatenate([w[half:d], w[d + half:2 * d]], axis=0).astype(jnp.bfloat16)

    def wcat(w):
        return jnp.concatenate([wlo(w), whi(w), w[2 * d:].astype(jnp.bfloat16)], axis=0)

    weights = (
        wcat(att_W1), att_b1[None, :],
        att_W2.astype(jnp.bfloat16), att_b2[None, :],
        att_W3.astype(jnp.bfloat16), att_b3[None, :],
        wcat(msg_W1), msg_b1[None, :],
        msg_W2.astype(jnp.bfloat16), msg_b2[None, :],
    )
    wspecs = [
        (3 * d, 96), (1, 96),
        (96, 48), (1, 48), (48, 1), (1, 1),
        (3 * d, 256), (1, 256),
        (256, d), (1, d),
    ]

    n_pad = NS * 8 * (-(-n // (NS * 8)))
    zeros_nd = jnp.zeros((n_pad, d), jnp.float32)

    def full(shape):
        return pl.BlockSpec(shape, lambda i: tuple(0 for _ in shape))

    nslices = 4
    bounds = [(tch * s) // nslices for s in range(nslices + 1)]
    slices = list(zip(bounds[:-1], bounds[1:]))
    accs = []
    for (c_lo, c_hi) in slices:
        nrows = (c_hi - c_lo) * CHUNK
        xg = _make_gather(n, d, c_lo, c_hi)(xpack, src, dst)

        be = 4000 if nrows % 4000 == 0 and (c_lo * CHUNK) % 4000 == 0 else CHUNK
        grid = nrows // be
        off = (c_lo * CHUNK) // be
        mask_from = None if ne >= c_hi * CHUNK else ne - c_lo * CHUNK

        m_arr = pl.pallas_call(
            functools.partial(_mlp_body, nrows, be, mask_from),
            grid=(grid,),
            in_specs=[
                pl.BlockSpec((be, d), lambda i: (i, 0)),
                pl.BlockSpec((be, d), lambda i, off=off: (i + off, 0)),
            ] + [full(s) for s in wspecs],
            out_specs=pl.BlockSpec((be, d), lambda i: (i, 0)),
            out_shape=jax.ShapeDtypeStruct((nrows, d), jnp.float32),
            compiler_params=pltpu.CompilerParams(
                dimension_semantics=("arbitrary",),
            ),
        )(xg, e_in, *weights)

        accs.append(_make_scatter(n_pad, d, c_lo, c_hi)(m_arr, dst, zeros_nd))

    bn = 2000
    x_out = pl.pallas_call(
        _combine_body,
        grid=(n // bn,),
        in_specs=[pl.BlockSpec((bn, d), lambda i: (i, 0))] + [
            pl.BlockSpec((NC, bn, d), lambda i: (0, i, 0)) for _ in accs
        ],
        out_specs=pl.BlockSpec((bn, d), lambda i: (i, 0)),
        out_shape=jax.ShapeDtypeStruct((n, d), jnp.float32),
    )(X, *accs)

    return (x_out, E)


# speedup vs baseline: 1.0114x; 1.0114x over previous
import functools

import jax
import jax.numpy as jnp
from jax import lax
from jax.experimental import pallas as pl
from jax.experimental.pallas import tpu as pltpu
from jax.experimental.pallas import tpu_sc as plsc

NC = 2
NS = 16
NW = NC * NS
CHUNK = 128


def _sc_mesh():
    return plsc.VectorSubcoreMesh(
        core_axis_name="c", subcore_axis_name="s", num_cores=NC, num_subcores=NS
    )


def _make_gather(n, d, c_lo, c_hi):
    dp = d // 2
    ncr = c_hi - c_lo

    @functools.partial(
        pl.kernel,
        mesh=_sc_mesh(),
        out_type=jax.ShapeDtypeStruct((ncr * CHUNK, d), jnp.int32),
        scratch_types=[
            pltpu.VMEM((CHUNK,), jnp.int32),
            pltpu.VMEM((CHUNK,), jnp.int32),
            pltpu.VMEM((CHUNK, dp), jnp.int32),
            pltpu.VMEM((CHUNK, dp), jnp.int32),
            pltpu.SemaphoreType.DMA,
            pltpu.SemaphoreType.DMA,
        ],
        compiler_params=pltpu.CompilerParams(use_tc_tiling_on_sc=False),
    )
    def gather_k(x_hbm, src_hbm, dst_hbm, xg_out, sidx, didx, srows, drows, s1, s2):
        cid = lax.axis_index("c")
        sid = lax.axis_index("s")
        wid = sid * NC + cid
        w0 = (ncr * wid) // NW
        w1 = (ncr * (wid + 1)) // NW

        def body(c, _):
            base = (c_lo + c) * CHUNK
            pltpu.sync_copy(src_hbm.at[pl.ds(base, CHUNK)], sidx)
            pltpu.sync_copy(dst_hbm.at[pl.ds(base, CHUNK)], didx)
            cp1 = pltpu.async_copy(x_hbm.at[sidx], srows, s1)
            cp2 = pltpu.async_copy(x_hbm.at[didx], drows, s2)
            cp1.wait()
            cp2.wait()
            pltpu.sync_copy(srows, xg_out.at[pl.ds(c * CHUNK, CHUNK), pl.ds(0, dp)])
            pltpu.sync_copy(drows, xg_out.at[pl.ds(c * CHUNK, CHUNK), pl.ds(dp, dp)])
            return 0

        lax.fori_loop(w0, w1, body, 0)

    return gather_k


def _make_scatter(n_pad, d, c_lo, c_hi):
    rows_per_tile = n_pad // NS
    ncr = c_hi - c_lo

    @functools.partial(
        pl.kernel,
        mesh=_sc_mesh(),
        out_type=jax.ShapeDtypeStruct((NC, n_pad, d), jnp.float32),
        scratch_types=[
            pltpu.VMEM_SHARED((n_pad, d), jnp.float32),
            pltpu.VMEM((CHUNK,), jnp.int32),
            pltpu.VMEM((CHUNK, d), jnp.float32),
        ],
    )
    def scatter_k(m_hbm, dst_hbm, zero_hbm, acc_out, acc, didx, mrows):
        cid = lax.axis_index("c")
        sid = lax.axis_index("s")
        wid = sid * NC + cid
        w0 = (ncr * wid) // NW
        w1 = (ncr * (wid + 1)) // NW

        r0 = sid * rows_per_tile
        pltpu.sync_copy(zero_hbm.at[pl.ds(r0, rows_per_tile)], acc.at[pl.ds(r0, rows_per_tile)])
        plsc.subcore_barrier()

        def body(c, _):
            pltpu.sync_copy(dst_hbm.at[pl.ds((c_lo + c) * CHUNK, CHUNK)], didx)
            pltpu.sync_copy(m_hbm.at[pl.ds(c * CHUNK, CHUNK)], mrows)
            pltpu.sync_copy(mrows, acc.at[didx], add=True)
            return 0

        lax.fori_loop(w0, w1, body, 0)
        plsc.subcore_barrier()
        pltpu.sync_copy(acc.at[pl.ds(r0, rows_per_tile)], acc_out.at[cid, pl.ds(r0, rows_per_tile)])

    return scatter_k


def _mlp_body(nrows, be, mask_from, xg_ref, e_ref,
              awlo_ref, awhi_ref, aw1e_ref, ab1_ref, aw2_ref, ab2_ref, aw3_ref, ab3_ref,
              mwlo_ref, mwhi_ref, mw1e_ref, mb1_ref, mw2_ref, mb2_ref, out_ref):
    xg = xg_ref[...]
    lo = jax.lax.bitcast_convert_type(xg << 16, jnp.float32).astype(jnp.bfloat16)
    hi = jax.lax.bitcast_convert_type(xg & jnp.int32(-65536), jnp.float32).astype(jnp.bfloat16)
    e = e_ref[...].astype(jnp.bfloat16)
    dot = functools.partial(jnp.dot, preferred_element_type=jnp.float32)
    t = dot(lo, awlo_ref[...]) + dot(hi, awhi_ref[...]) + dot(e, aw1e_ref[...]) + ab1_ref[...]
    t = jnp.maximum(t, 0.0).astype(jnp.bfloat16)
    t = jnp.maximum(dot(t, aw2_ref[...]) + ab2_ref[...], 0.0)
    a = dot(t.astype(jnp.bfloat16), aw3_ref[...]) + ab3_ref[...]
    h = dot(lo, mwlo_ref[...]) + dot(hi, mwhi_ref[...]) + dot(e, mw1e_ref[...]) + mb1_ref[...]
    h = jnp.maximum(h, 0.0).astype(jnp.bfloat16)
    m = dot(h, mw2_ref[...]) + mb2_ref[...]
    msg = jax.nn.sigmoid(a) * m
    if mask_from is not None:
        row = pl.program_id(0) * be + lax.broadcasted_iota(jnp.int32, msg.shape, 0)
        msg = jnp.where(row < mask_from, msg, 0.0)
    out_ref[...] = msg


def _combine_body(x_ref, *refs):
    arefs = refs[:-1]
    out_ref = refs[-1]
    acc = x_ref[...]
    for a_ref in arefs:
        acc = acc + a_ref[0] + a_ref[1]
    out_ref[...] = acc


def kernel(X, E, emb_nodes, emb_edges, edge_index,
           att_W1, att_b1, att_W2, att_b2, att_W3, att_b3,
           msg_W1, msg_b1, msg_W2, msg_b2):
    n, d = X.shape
    ne = E.shape[0]
    tch = -(-ne // CHUNK)
    ne_c = tch * CHUNK
    pad = ne_c - ne

    if pad:
        src = jnp.concatenate([edge_index[0], jnp.zeros((pad,), jnp.int32)])
        dst = jnp.concatenate([edge_index[1], jnp.zeros((pad,), jnp.int32)])
        e_in = jnp.concatenate([E, jnp.zeros((pad, d), jnp.float32)], axis=0)
    else:
        src = edge_index[0]
        dst = edge_index[1]
        e_in = E
    e_ret = jnp.copy(E)

    xb = X.astype(jnp.bfloat16)
    lo16 = jax.lax.bitcast_convert_type(xb[:, : d // 2], jnp.uint16).astype(jnp.uint32)
    hi16 = jax.lax.bitcast_convert_type(xb[:, d // 2:], jnp.uint16).astype(jnp.uint32)
    xpack = jax.lax.bitcast_convert_type((hi16 << 16) | lo16, jnp.int32)

    half = d // 2

    def wlo(w):
        return jnp.concatenate([w[:half], w[d:d + half]], axis=0).astype(jnp.bfloat16)

    def whi(w):
        return jnp.concatenate([w[half:d], w[d + half:2 * d]], axis=0).astype(jnp.bfloat16)

    weights = (
        wlo(att_W1), whi(att_W1), att_W1[2 * d:].astype(jnp.bfloat16), att_b1[None, :],
        att_W2.astype(jnp.bfloat16), att_b2[None, :],
        att_W3.astype(jnp.bfloat16), att_b3[None, :],
        wlo(msg_W1), whi(msg_W1), msg_W1[2 * d:].astype(jnp.bfloat16), msg_b1[None, :],
        msg_W2.astype(jnp.bfloat16), msg_b2[None, :],
    )
    wspecs = [
        (d, 96), (d, 96), (d, 96), (1, 96),
        (96, 48), (1, 48), (48, 1), (1, 1),
        (d, 256), (d, 256), (d, 256), (1, 256),
        (256, d), (1, d),
    ]

    n_pad = NS * 8 * (-(-n // (NS * 8)))
    zeros_nd = jnp.zeros((n_pad, d), jnp.float32)

    def full(shape):
        return pl.BlockSpec(shape, lambda i: tuple(0 for _ in shape))

    nslices = 4
    bounds = [(tch * s) // nslices for s in range(nslices + 1)]
    slices = list(zip(bounds[:-1], bounds[1:]))
    accs = []
    for (c_lo, c_hi) in slices:
        nrows = (c_hi - c_lo) * CHUNK
        xg = _make_gather(n, d, c_lo, c_hi)(xpack, src, dst)

        be = 4000 if nrows % 4000 == 0 and (c_lo * CHUNK) % 4000 == 0 else CHUNK
        grid = nrows // be
        off = (c_lo * CHUNK) // be
        mask_from = None if ne >= c_hi * CHUNK else ne - c_lo * CHUNK

        m_arr = pl.pallas_call(
            functools.partial(_mlp_body, nrows, be, mask_from),
            grid=(grid,),
            in_specs=[
                pl.BlockSpec((be, d), lambda i: (i, 0)),
                pl.BlockSpec((be, d), lambda i, off=off: (i + off, 0)),
            ] + [full(s) for s in wspecs],
            out_specs=pl.BlockSpec((be, d), lambda i: (i, 0)),
            out_shape=jax.ShapeDtypeStruct((nrows, d), jnp.float32),
            compiler_params=pltpu.CompilerParams(
                dimension_semantics=("arbitrary",),
            ),
        )(xg, e_in, *weights)

        accs.append(_make_scatter(n_pad, d, c_lo, c_hi)(m_arr, dst, zeros_nd))

    bn = 2000
    x_out = pl.pallas_call(
        _combine_body,
        grid=(n // bn,),
        in_specs=[pl.BlockSpec((bn, d), lambda i: (i, 0))] + [
            pl.BlockSpec((NC, bn, d), lambda i: (0, i, 0)) for _ in accs
        ],
        out_specs=pl.BlockSpec((bn, d), lambda i: (i, 0)),
        out_shape=jax.ShapeDtypeStruct((n, d), jnp.float32),
    )(X, *accs)

    return (x_out, e_ret)
